# baseline (speedup 1.0000x reference)
import functools

import jax
import jax.numpy as jnp
from jax.experimental import pallas as pl
from jax.experimental.pallas import tpu as pltpu

NT = 256


def _kernel(vrow, words_smem, w_hbm, wtail_ref, wcol_ref, root_ref, out_ref,
            g0, g1, gt, tile, troot, ttail, dsem):
    b = pl.program_id(0)
    t = pl.program_id(1)
    ntiles = pl.num_programs(1)
    n = tile.shape[0]
    vw = gt.shape[0]
    gstride = vw // 128
    vmain = (vrow // 128) * 128

    def issue_rows(g_ref, sem, tt):
        base = tt * NT

        def body(i, _):
            r = words_smem[b, base + i]
            pltpu.make_async_copy(
                w_hbm.at[pl.ds(r, 1), pl.ds(0, vmain)],
                g_ref.at[i, :, pl.ds(0, vmain)],
                sem,
            ).start()
            return 0

        jax.lax.fori_loop(0, NT, body, 0)

    def wait_rows(g_ref, sem):
        sl = g_ref.at[:, :, pl.ds(0, vmain)]
        pltpu.make_async_copy(sl, sl, sem).wait()

    slot = jax.lax.rem(t, 2)

    @pl.when(t == 0)
    def _():
        issue_rows(g0, dsem.at[0], 0)

    @pl.when(slot == 0)
    def _():
        wait_rows(g0, dsem.at[0])

    @pl.when(slot == 1)
    def _():
        wait_rows(g1, dsem.at[1])

    @pl.when(jnp.logical_and(t + 1 < ntiles, slot == 0))
    def _():
        issue_rows(g1, dsem.at[1], t + 1)

    @pl.when(jnp.logical_and(t + 1 < ntiles, slot == 1))
    def _():
        issue_rows(g0, dsem.at[0], t + 1)

    def do_transpose(g_ref):
        g2 = g_ref.reshape(NT * gstride, 128)
        for q in range(vmain // 128):
            col = g2[q:q + NT * gstride:gstride, :]
            gt[pl.ds(q * 128, 128), :] = col.T

    @pl.when(slot == 0)
    def _():
        do_transpose(g0)

    @pl.when(slot == 1)
    def _():
        do_transpose(g1)

    def pt_body(io, _):
        i0 = io * 8
        rows = []
        for u in range(8):
            r = words_smem[b, t * NT + i0 + u]
            r8 = pl.multiple_of((r >> 3) << 3, 8)
            chunk = wtail_ref[pl.ds(r8, 8), :]
            rolled = pltpu.roll(chunk, 8 - (r & 7), axis=0)
            rows.append(rolled[0:1, :])
        ttail[pl.ds(i0, 8), :] = jnp.concatenate(rows, axis=0)
        return 0

    jax.lax.fori_loop(0, NT // 8, pt_body, 0)
    gt[pl.ds(vrow - 128, 128), :] = ttail[:, :].T

    def mg_body(mo, _):
        m0 = mo * 8
        rows = []
        for u in range(8):
            c = words_smem[b, m0 + u]
            c8 = pl.multiple_of((c >> 3) << 3, 8)
            chunk = gt[pl.ds(c8, 8), :]
            rolled = pltpu.roll(chunk, 8 - (c & 7), axis=0)
            rows.append(rolled[0:1, :])
        tile[pl.ds(m0, 8), :] = jnp.concatenate(rows, axis=0)
        return 0

    jax.lax.fori_loop(0, n // 8, mg_body, 0)

    def rg_body(io, _):
        for u in range(8):
            i = io * 8 + u
            q = words_smem[b, t * NT + i] >> 7
            troot[i, 0] = root_ref[q, 0]
        return 0

    jax.lax.fori_loop(0, NT // 8, rg_body, 0)

    o_blk = tile[:, :].T
    rvals = wcol_ref[0, 0]
    lane = jax.lax.broadcasted_iota(jnp.int32, (NT, 128), 1)
    tr = troot[:, 0, :]
    rv = jnp.sum(jnp.where(lane == (rvals & 127), tr, 0.0),
                 axis=1, keepdims=True)
    row = jax.lax.broadcasted_iota(jnp.int32, (NT, n), 0)
    col = jax.lax.broadcasted_iota(jnp.int32, (NT, n), 1)
    diag = col == t * NT + row
    out_ref[0] = o_blk + jnp.where(diag, jnp.broadcast_to(rv, (NT, n)), 0.0)


def kernel(words, W, root):
    B, N = words.shape
    V = W.shape[0]
    ntiles = N // NT
    vw = ((V + 127) // 128) * 128
    words = words.astype(jnp.int32)
    rootp = jnp.pad(root, (0, vw - V)).reshape(vw // 128, 1, 128)
    wcol4 = words.reshape(B, ntiles, NT, 1)
    wtail = W[:, V - 128:]

    grid_spec = pltpu.PrefetchScalarGridSpec(
        num_scalar_prefetch=1,
        grid=(B, ntiles),
        in_specs=[
            pl.BlockSpec(memory_space=pl.ANY),
            pl.BlockSpec((V, 128), lambda b, t, w: (0, 0)),
            pl.BlockSpec((1, 1, NT, 1), lambda b, t, w: (b, t, 0, 0)),
            pl.BlockSpec((vw // 128, 1, 128), lambda b, t, w: (0, 0, 0)),
        ],
        out_specs=pl.BlockSpec((1, NT, N), lambda b, t, w: (b, t, 0)),
        scratch_shapes=[
            pltpu.VMEM((NT, 1, vw), jnp.float32),
            pltpu.VMEM((NT, 1, vw), jnp.float32),
            pltpu.VMEM((vw, NT), jnp.float32),
            pltpu.VMEM((N, NT), jnp.float32),
            pltpu.VMEM((NT, 1, 128), jnp.float32),
            pltpu.VMEM((NT, 128), jnp.float32),
            pltpu.SemaphoreType.DMA((2,)),
        ],
    )
    return pl.pallas_call(
        functools.partial(_kernel, V),
        out_shape=jax.ShapeDtypeStruct((B, N, N), jnp.float32),
        grid_spec=grid_spec,
        compiler_params=pltpu.CompilerParams(
            dimension_semantics=("parallel", "arbitrary"),
            vmem_limit_bytes=56 * 1024 * 1024,
        ),
        name="gather_bilinear",
    )(words, W, wtail, wcol4, rootp)


# speedup vs baseline: 1.2224x; 1.2224x over previous
import functools

import jax
import jax.numpy as jnp
from jax.experimental import pallas as pl
from jax.experimental.pallas import tpu as pltpu

NT = 256
MG_U = 8


def _kernel(vrow, words_smem, w_hbm, wtail_ref, wcol_ref, root_ref, out_ref,
            g0, g1, gt, tile, troot, ttail, dsem):
    b = pl.program_id(0)
    t = pl.program_id(1)
    ntiles = pl.num_programs(1)
    n = tile.shape[0]
    vw = gt.shape[0]
    gstride = vw // 128
    vmain = (vrow // 128) * 128
    half = NT // 128

    def issue_rows(g_ref, sem, tt):
        base = tt * NT

        def body(i, _):
            r = words_smem[b, base + i]
            pltpu.make_async_copy(
                w_hbm.at[pl.ds(r, 1), pl.ds(0, vmain)],
                g_ref.at[i, :, pl.ds(0, vmain)],
                sem,
            ).start()
            return 0

        jax.lax.fori_loop(0, NT, body, 0)

    def wait_rows(g_ref, sem):
        sl = g_ref.at[:, :, pl.ds(0, vmain)]
        pltpu.make_async_copy(sl, sl, sem).wait()

    slot = jax.lax.rem(t, 2)

    @pl.when(t == 0)
    def _():
        issue_rows(g0, dsem.at[0], 0)

    @pl.when(slot == 0)
    def _():
        wait_rows(g0, dsem.at[0])

    @pl.when(slot == 1)
    def _():
        wait_rows(g1, dsem.at[1])

    @pl.when(jnp.logical_and(t + 1 < ntiles, slot == 0))
    def _():
        issue_rows(g1, dsem.at[1], t + 1)

    @pl.when(jnp.logical_and(t + 1 < ntiles, slot == 1))
    def _():
        issue_rows(g0, dsem.at[0], t + 1)

    gt_view = gt.reshape(vw * half, 128)

    def store_rows(c0, block):
        base0 = c0 * half
        for h in range(half):
            gt_view[base0 + h:base0 + h + 128 * half:half, :] = (
                block[:, 128 * h:128 * (h + 1)])

    def do_transpose(g_ref):
        g2 = g_ref.reshape(NT * gstride, 128)
        for q in range(vmain // 128):
            col = g2[q:q + NT * gstride:gstride, :]
            store_rows(q * 128, col.T)

    @pl.when(slot == 0)
    def _():
        do_transpose(g0)

    @pl.when(slot == 1)
    def _():
        do_transpose(g1)

    def pt_body(io, _):
        i0 = io * MG_U
        rows = []
        for u in range(MG_U):
            r = words_smem[b, t * NT + i0 + u]
            r8 = pl.multiple_of((r >> 3) << 3, 8)
            chunk = wtail_ref[pl.ds(r8, 8), :]
            rolled = pltpu.roll(chunk, 8 - (r & 7), axis=0)
            rows.append(rolled[0:1, :])
        ttail[pl.ds(i0, 8), :] = jnp.concatenate(rows, axis=0)
        return 0

    jax.lax.fori_loop(0, NT // MG_U, pt_body, 0)
    store_rows(vrow - 128, ttail[:, :].T)

    def mg_body(mo, _):
        m0 = mo * MG_U
        for u in range(MG_U):
            c = words_smem[b, m0 + u]
            tile[m0 + u, 0] = gt[c, 0]
        return 0

    jax.lax.fori_loop(0, n // MG_U, mg_body, 0)

    def rg_body(io, _):
        for u in range(MG_U):
            i = io * MG_U + u
            q = words_smem[b, t * NT + i] >> 7
            troot[i, 0] = root_ref[q, 0]
        return 0

    jax.lax.fori_loop(0, NT // MG_U, rg_body, 0)

    rvals = wcol_ref[0, 0]
    lane = jax.lax.broadcasted_iota(jnp.int32, (NT, 128), 1)
    tr = troot[:, 0, :]
    rv = jnp.sum(jnp.where(lane == (rvals & 127), tr, 0.0),
                 axis=1, keepdims=True)

    tile_view = tile.reshape(n * half, 128)
    for h in range(half):
        colf = tile_view[h:h + n * half:half, :]
        o_blk = colf.T
        row = jax.lax.broadcasted_iota(jnp.int32, (128, n), 0)
        col = jax.lax.broadcasted_iota(jnp.int32, (128, n), 1)
        diag = col == t * NT + 128 * h + row
        rv_h = jax.lax.slice_in_dim(rv, 128 * h, 128 * (h + 1), axis=0)
        out_ref[0, pl.ds(128 * h, 128), :] = o_blk + jnp.where(
            diag, jnp.broadcast_to(rv_h, (128, n)), 0.0)


def kernel(words, W, root):
    B, N = words.shape
    V = W.shape[0]
    ntiles = N // NT
    vw = ((V + 127) // 128) * 128
    words = words.astype(jnp.int32)
    rootp = jnp.pad(root, (0, vw - V)).reshape(vw // 128, 1, 128)
    wcol4 = words.reshape(B, ntiles, NT, 1)
    wtail = W[:, V - 128:]

    grid_spec = pltpu.PrefetchScalarGridSpec(
        num_scalar_prefetch=1,
        grid=(B, ntiles),
        in_specs=[
            pl.BlockSpec(memory_space=pl.ANY),
            pl.BlockSpec((V, 128), lambda b, t, w: (0, 0)),
            pl.BlockSpec((1, 1, NT, 1), lambda b, t, w: (b, t, 0, 0)),
            pl.BlockSpec((vw // 128, 1, 128), lambda b, t, w: (0, 0, 0)),
        ],
        out_specs=pl.BlockSpec((1, NT, N), lambda b, t, w: (b, t, 0)),
        scratch_shapes=[
            pltpu.VMEM((NT, 1, vw), jnp.float32),
            pltpu.VMEM((NT, 1, vw), jnp.float32),
            pltpu.VMEM((vw, 1, NT), jnp.float32),
            pltpu.VMEM((N, 1, NT), jnp.float32),
            pltpu.VMEM((NT, 1, 128), jnp.float32),
            pltpu.VMEM((NT, 128), jnp.float32),
            pltpu.SemaphoreType.DMA((2,)),
        ],
    )
    return pl.pallas_call(
        functools.partial(_kernel, V),
        out_shape=jax.ShapeDtypeStruct((B, N, N), jnp.float32),
        grid_spec=grid_spec,
        compiler_params=pltpu.CompilerParams(
            dimension_semantics=("parallel", "arbitrary"),
            vmem_limit_bytes=56 * 1024 * 1024,
        ),
        name="gather_bilinear",
    )(words, W, wtail, wcol4, rootp)


# speedup vs baseline: 1.4382x; 1.1765x over previous
import functools

import jax
import jax.numpy as jnp
from jax.experimental import pallas as pl
from jax.experimental.pallas import tpu as pltpu

NT = 256
MG_U = 16
DMA_U = 16


def _kernel(vrow, words_smem, w_hbm, wtail_ref, wcol_ref, root_ref, out_ref,
            g0, g1, gt, tile, troot, ttail, dsem):
    b = pl.program_id(0)
    t = pl.program_id(1)
    ntiles = pl.num_programs(1)
    n = tile.shape[0]
    vw = gt.shape[0]
    gstride = vw // 128
    vmain = (vrow // 128) * 128
    half = NT // 128

    def issue_rows(g_ref, sem, tt):
        base = tt * NT

        def body(io, _):
            i0 = io * DMA_U
            for u in range(DMA_U):
                i = i0 + u
                r = words_smem[b, base + i]
                pltpu.make_async_copy(
                    w_hbm.at[pl.ds(r, 1), pl.ds(0, vmain)],
                    g_ref.at[i, :, pl.ds(0, vmain)],
                    sem,
                ).start()
            return 0

        jax.lax.fori_loop(0, NT // DMA_U, body, 0)

    def wait_rows(g_ref, sem):
        sl = g_ref.at[:, :, pl.ds(0, vmain)]
        pltpu.make_async_copy(sl, sl, sem).wait()

    slot = jax.lax.rem(t, 2)

    @pl.when(t == 0)
    def _():
        issue_rows(g0, dsem.at[0], 0)

    @pl.when(slot == 0)
    def _():
        wait_rows(g0, dsem.at[0])

    @pl.when(slot == 1)
    def _():
        wait_rows(g1, dsem.at[1])

    @pl.when(jnp.logical_and(t + 1 < ntiles, slot == 0))
    def _():
        issue_rows(g1, dsem.at[1], t + 1)

    @pl.when(jnp.logical_and(t + 1 < ntiles, slot == 1))
    def _():
        issue_rows(g0, dsem.at[0], t + 1)

    gt_view = gt.reshape(vw * half, 128)

    def store_rows(c0, block):
        base0 = c0 * half
        for h in range(half):
            gt_view[base0 + h:base0 + h + 128 * half:half, :] = (
                block[:, 128 * h:128 * (h + 1)])

    def do_transpose(g_ref):
        g2 = g_ref.reshape(NT * gstride, 128)
        for q in range(vmain // 128):
            col = g2[q:q + NT * gstride:gstride, :]
            store_rows(q * 128, col.T)

    @pl.when(slot == 0)
    def _():
        do_transpose(g0)

    @pl.when(slot == 1)
    def _():
        do_transpose(g1)

    def pt_body(io, _):
        i0 = io * MG_U
        rows = []
        for u in range(MG_U):
            r = words_smem[b, t * NT + i0 + u]
            r8 = pl.multiple_of((r >> 3) << 3, 8)
            chunk = wtail_ref[pl.ds(r8, 8), :]
            rolled = pltpu.roll(chunk, 8 - (r & 7), axis=0)
            rows.append(rolled[0:1, :])
        ttail[pl.ds(i0, MG_U), :] = jnp.concatenate(rows, axis=0)
        return 0

    jax.lax.fori_loop(0, NT // MG_U, pt_body, 0)
    store_rows(vrow - 128, ttail[:, :].T)

    def mg_body(mo, _):
        m0 = mo * MG_U
        for u in range(MG_U):
            c = words_smem[b, m0 + u]
            tile[m0 + u, 0] = gt[c, 0]
        return 0

    jax.lax.fori_loop(0, n // MG_U, mg_body, 0)

    def rg_body(io, _):
        for u in range(MG_U):
            i = io * MG_U + u
            q = words_smem[b, t * NT + i] >> 7
            troot[i, 0] = root_ref[q, 0]
        return 0

    jax.lax.fori_loop(0, NT // MG_U, rg_body, 0)

    rvals = wcol_ref[0, 0]
    lane = jax.lax.broadcasted_iota(jnp.int32, (NT, 128), 1)
    tr = troot[:, 0, :]
    rv = jnp.sum(jnp.where(lane == (rvals & 127), tr, 0.0),
                 axis=1, keepdims=True)

    tile_view = tile.reshape(n * half, 128)
    for h in range(half):
        colf = tile_view[h:h + n * half:half, :]
        o_blk = colf.T
        row = jax.lax.broadcasted_iota(jnp.int32, (128, n), 0)
        col = jax.lax.broadcasted_iota(jnp.int32, (128, n), 1)
        diag = col == t * NT + 128 * h + row
        rv_h = jax.lax.slice_in_dim(rv, 128 * h, 128 * (h + 1), axis=0)
        out_ref[0, pl.ds(128 * h, 128), :] = o_blk + jnp.where(
            diag, jnp.broadcast_to(rv_h, (128, n)), 0.0)


def kernel(words, W, root):
    B, N = words.shape
    V = W.shape[0]
    ntiles = N // NT
    vw = ((V + 127) // 128) * 128
    words = words.astype(jnp.int32)
    rootp = jnp.pad(root, (0, vw - V)).reshape(vw // 128, 1, 128)
    wcol4 = words.reshape(B, ntiles, NT, 1)
    wtail = W[:, V - 128:]

    grid_spec = pltpu.PrefetchScalarGridSpec(
        num_scalar_prefetch=1,
        grid=(B, ntiles),
        in_specs=[
            pl.BlockSpec(memory_space=pl.ANY),
            pl.BlockSpec((V, 128), lambda b, t, w: (0, 0)),
            pl.BlockSpec((1, 1, NT, 1), lambda b, t, w: (b, t, 0, 0)),
            pl.BlockSpec((vw // 128, 1, 128), lambda b, t, w: (0, 0, 0)),
        ],
        out_specs=pl.BlockSpec((1, NT, N), lambda b, t, w: (b, t, 0)),
        scratch_shapes=[
            pltpu.VMEM((NT, 1, vw), jnp.float32),
            pltpu.VMEM((NT, 1, vw), jnp.float32),
            pltpu.VMEM((vw, 1, NT), jnp.float32),
            pltpu.VMEM((N, 1, NT), jnp.float32),
            pltpu.VMEM((NT, 1, 128), jnp.float32),
            pltpu.VMEM((NT, 128), jnp.float32),
            pltpu.SemaphoreType.DMA((2,)),
        ],
    )
    return pl.pallas_call(
        functools.partial(_kernel, V),
        out_shape=jax.ShapeDtypeStruct((B, N, N), jnp.float32),
        grid_spec=grid_spec,
        compiler_params=pltpu.CompilerParams(
            dimension_semantics=("parallel", "arbitrary"),
            vmem_limit_bytes=56 * 1024 * 1024,
        ),
        name="gather_bilinear",
    )(words, W, wtail, wcol4, rootp)


# speedup vs baseline: 1.4979x; 1.0415x over previous
import functools

import jax
import jax.numpy as jnp
from jax.experimental import pallas as pl
from jax.experimental.pallas import tpu as pltpu

NT = 256
MG_U = 32
DMA_U = 32


def _kernel(vrow, words_smem, w_hbm, wtail_ref, wcol_ref, root_ref, out_ref,
            g0, g1, gt, tile, troot, ttail, dsem):
    b = pl.program_id(0)
    t = pl.program_id(1)
    ntiles = pl.num_programs(1)
    n = tile.shape[0]
    vw = gt.shape[0]
    gstride = vw // 128
    vmain = (vrow // 128) * 128
    half = NT // 128

    def issue_rows(g_ref, sem, tt):
        base = tt * NT

        def body(io, _):
            i0 = io * DMA_U
            for u in range(DMA_U):
                i = i0 + u
                r = words_smem[b, base + i]
                pltpu.make_async_copy(
                    w_hbm.at[pl.ds(r, 1), pl.ds(0, vmain)],
                    g_ref.at[i, :, pl.ds(0, vmain)],
                    sem,
                ).start()
            return 0

        jax.lax.fori_loop(0, NT // DMA_U, body, 0)

    def wait_rows(g_ref, sem):
        sl = g_ref.at[:, :, pl.ds(0, vmain)]
        pltpu.make_async_copy(sl, sl, sem).wait()

    slot = jax.lax.rem(t, 2)

    @pl.when(t == 0)
    def _():
        issue_rows(g0, dsem.at[0], 0)

    @pl.when(slot == 0)
    def _():
        wait_rows(g0, dsem.at[0])

    @pl.when(slot == 1)
    def _():
        wait_rows(g1, dsem.at[1])

    @pl.when(jnp.logical_and(t + 1 < ntiles, slot == 0))
    def _():
        issue_rows(g1, dsem.at[1], t + 1)

    @pl.when(jnp.logical_and(t + 1 < ntiles, slot == 1))
    def _():
        issue_rows(g0, dsem.at[0], t + 1)

    gt_view = gt.reshape(vw * half, 128)

    def store_rows(c0, block):
        base0 = c0 * half
        for h in range(half):
            gt_view[base0 + h:base0 + h + 128 * half:half, :] = (
                block[:, 128 * h:128 * (h + 1)])

    def do_transpose(g_ref):
        g2 = g_ref.reshape(NT * gstride, 128)
        for q in range(vmain // 128):
            col = g2[q:q + NT * gstride:gstride, :]
            store_rows(q * 128, col.T)

    @pl.when(slot == 0)
    def _():
        do_transpose(g0)

    @pl.when(slot == 1)
    def _():
        do_transpose(g1)

    def pt_body(io, _):
        i0 = io * MG_U
        rows = []
        for u in range(MG_U):
            r = words_smem[b, t * NT + i0 + u]
            r8 = pl.multiple_of((r >> 3) << 3, 8)
            chunk = wtail_ref[pl.ds(r8, 8), :]
            rolled = pltpu.roll(chunk, 8 - (r & 7), axis=0)
            rows.append(rolled[0:1, :])
        ttail[pl.ds(i0, MG_U), :] = jnp.concatenate(rows, axis=0)
        return 0

    jax.lax.fori_loop(0, NT // MG_U, pt_body, 0)
    store_rows(vrow - 128, ttail[:, :].T)

    def mg_body(mo, _):
        m0 = mo * MG_U
        for u in range(MG_U):
            c = words_smem[b, m0 + u]
            tile[m0 + u, 0] = gt[c, 0]
        return 0

    jax.lax.fori_loop(0, n // MG_U, mg_body, 0)

    def rg_body(io, _):
        for u in range(MG_U):
            i = io * MG_U + u
            q = words_smem[b, t * NT + i] >> 7
            troot[i, 0] = root_ref[q, 0]
        return 0

    jax.lax.fori_loop(0, NT // MG_U, rg_body, 0)

    rvals = wcol_ref[0, 0]
    lane = jax.lax.broadcasted_iota(jnp.int32, (NT, 128), 1)
    tr = troot[:, 0, :]
    rv = jnp.sum(jnp.where(lane == (rvals & 127), tr, 0.0),
                 axis=1, keepdims=True)

    tile_view = tile.reshape(n * half, 128)
    for h in range(half):
        colf = tile_view[h:h + n * half:half, :]
        o_blk = colf.T
        row = jax.lax.broadcasted_iota(jnp.int32, (128, n), 0)
        col = jax.lax.broadcasted_iota(jnp.int32, (128, n), 1)
        diag = col == t * NT + 128 * h + row
        rv_h = jax.lax.slice_in_dim(rv, 128 * h, 128 * (h + 1), axis=0)
        out_ref[0, pl.ds(128 * h, 128), :] = o_blk + jnp.where(
            diag, jnp.broadcast_to(rv_h, (128, n)), 0.0)


def kernel(words, W, root):
    B, N = words.shape
    V = W.shape[0]
    ntiles = N // NT
    vw = ((V + 127) // 128) * 128
    words = words.astype(jnp.int32)
    rootp = jnp.pad(root, (0, vw - V)).reshape(vw // 128, 1, 128)
    wcol4 = words.reshape(B, ntiles, NT, 1)
    wtail = W[:, V - 128:]

    grid_spec = pltpu.PrefetchScalarGridSpec(
        num_scalar_prefetch=1,
        grid=(B, ntiles),
        in_specs=[
            pl.BlockSpec(memory_space=pl.ANY),
            pl.BlockSpec((V, 128), lambda b, t, w: (0, 0)),
            pl.BlockSpec((1, 1, NT, 1), lambda b, t, w: (b, t, 0, 0)),
            pl.BlockSpec((vw // 128, 1, 128), lambda b, t, w: (0, 0, 0)),
        ],
        out_specs=pl.BlockSpec((1, NT, N), lambda b, t, w: (b, t, 0)),
        scratch_shapes=[
            pltpu.VMEM((NT, 1, vw), jnp.float32),
            pltpu.VMEM((NT, 1, vw), jnp.float32),
            pltpu.VMEM((vw, 1, NT), jnp.float32),
            pltpu.VMEM((N, 1, NT), jnp.float32),
            pltpu.VMEM((NT, 1, 128), jnp.float32),
            pltpu.VMEM((NT, 128), jnp.float32),
            pltpu.SemaphoreType.DMA((2,)),
        ],
    )
    return pl.pallas_call(
        functools.partial(_kernel, V),
        out_shape=jax.ShapeDtypeStruct((B, N, N), jnp.float32),
        grid_spec=grid_spec,
        compiler_params=pltpu.CompilerParams(
            dimension_semantics=("parallel", "arbitrary"),
            vmem_limit_bytes=56 * 1024 * 1024,
        ),
        name="gather_bilinear",
    )(words, W, wtail, wcol4, rootp)


# speedup vs baseline: 1.5291x; 1.0208x over previous
import functools

import jax
import jax.numpy as jnp
from jax.experimental import pallas as pl
from jax.experimental.pallas import tpu as pltpu

NT = 256
MG_U = 64
DMA_U = 64


def _kernel(vrow, words_smem, w_hbm, wtail_ref, wcol_ref, root_ref, out_ref,
            g0, g1, gt, tile, troot, ttail, dsem):
    b = pl.program_id(0)
    t = pl.program_id(1)
    ntiles = pl.num_programs(1)
    n = tile.shape[0]
    vw = gt.shape[0]
    gstride = vw // 128
    vmain = (vrow // 128) * 128
    half = NT // 128

    def issue_rows(g_ref, sem, tt):
        base = tt * NT

        def body(io, _):
            i0 = io * DMA_U
            for u in range(DMA_U):
                i = i0 + u
                r = words_smem[b, base + i]
                pltpu.make_async_copy(
                    w_hbm.at[pl.ds(r, 1), pl.ds(0, vmain)],
                    g_ref.at[i, :, pl.ds(0, vmain)],
                    sem,
                ).start()
            return 0

        jax.lax.fori_loop(0, NT // DMA_U, body, 0)

    def wait_rows(g_ref, sem):
        sl = g_ref.at[:, :, pl.ds(0, vmain)]
        pltpu.make_async_copy(sl, sl, sem).wait()

    slot = jax.lax.rem(t, 2)

    @pl.when(t == 0)
    def _():
        issue_rows(g0, dsem.at[0], 0)

    @pl.when(slot == 0)
    def _():
        wait_rows(g0, dsem.at[0])

    @pl.when(slot == 1)
    def _():
        wait_rows(g1, dsem.at[1])

    @pl.when(jnp.logical_and(t + 1 < ntiles, slot == 0))
    def _():
        issue_rows(g1, dsem.at[1], t + 1)

    @pl.when(jnp.logical_and(t + 1 < ntiles, slot == 1))
    def _():
        issue_rows(g0, dsem.at[0], t + 1)

    gt_view = gt.reshape(vw * half, 128)

    def store_rows(c0, block):
        base0 = c0 * half
        for h in range(half):
            gt_view[base0 + h:base0 + h + 128 * half:half, :] = (
                block[:, 128 * h:128 * (h + 1)])

    def do_transpose(g_ref):
        g2 = g_ref.reshape(NT * gstride, 128)
        for q in range(vmain // 128):
            col = g2[q:q + NT * gstride:gstride, :]
            store_rows(q * 128, col.T)

    @pl.when(slot == 0)
    def _():
        do_transpose(g0)

    @pl.when(slot == 1)
    def _():
        do_transpose(g1)

    def pt_body(io, _):
        i0 = io * MG_U
        rows = []
        for u in range(MG_U):
            r = words_smem[b, t * NT + i0 + u]
            r8 = pl.multiple_of((r >> 3) << 3, 8)
            chunk = wtail_ref[pl.ds(r8, 8), :]
            rolled = pltpu.roll(chunk, 8 - (r & 7), axis=0)
            rows.append(rolled[0:1, :])
        ttail[pl.ds(i0, MG_U), :] = jnp.concatenate(rows, axis=0)
        return 0

    jax.lax.fori_loop(0, NT // MG_U, pt_body, 0)
    store_rows(vrow - 128, ttail[:, :].T)

    def mg_body(mo, _):
        m0 = mo * MG_U
        for u in range(MG_U):
            c = words_smem[b, m0 + u]
            tile[m0 + u, 0] = gt[c, 0]
        return 0

    jax.lax.fori_loop(0, n // MG_U, mg_body, 0)

    def rg_body(io, _):
        for u in range(MG_U):
            i = io * MG_U + u
            q = words_smem[b, t * NT + i] >> 7
            troot[i, 0] = root_ref[q, 0]
        return 0

    jax.lax.fori_loop(0, NT // MG_U, rg_body, 0)

    rvals = wcol_ref[0, 0]
    lane = jax.lax.broadcasted_iota(jnp.int32, (NT, 128), 1)
    tr = troot[:, 0, :]
    rv = jnp.sum(jnp.where(lane == (rvals & 127), tr, 0.0),
                 axis=1, keepdims=True)

    tile_view = tile.reshape(n * half, 128)
    for h in range(half):
        colf = tile_view[h:h + n * half:half, :]
        o_blk = colf.T
        row = jax.lax.broadcasted_iota(jnp.int32, (128, n), 0)
        col = jax.lax.broadcasted_iota(jnp.int32, (128, n), 1)
        diag = col == t * NT + 128 * h + row
        rv_h = jax.lax.slice_in_dim(rv, 128 * h, 128 * (h + 1), axis=0)
        out_ref[0, pl.ds(128 * h, 128), :] = o_blk + jnp.where(
            diag, jnp.broadcast_to(rv_h, (128, n)), 0.0)


def kernel(words, W, root):
    B, N = words.shape
    V = W.shape[0]
    ntiles = N // NT
    vw = ((V + 127) // 128) * 128
    words = words.astype(jnp.int32)
    rootp = jnp.pad(root, (0, vw - V)).reshape(vw // 128, 1, 128)
    wcol4 = words.reshape(B, ntiles, NT, 1)
    wtail = W[:, V - 128:]

    grid_spec = pltpu.PrefetchScalarGridSpec(
        num_scalar_prefetch=1,
        grid=(B, ntiles),
        in_specs=[
            pl.BlockSpec(memory_space=pl.ANY),
            pl.BlockSpec((V, 128), lambda b, t, w: (0, 0)),
            pl.BlockSpec((1, 1, NT, 1), lambda b, t, w: (b, t, 0, 0)),
            pl.BlockSpec((vw // 128, 1, 128), lambda b, t, w: (0, 0, 0)),
        ],
        out_specs=pl.BlockSpec((1, NT, N), lambda b, t, w: (b, t, 0)),
        scratch_shapes=[
            pltpu.VMEM((NT, 1, vw), jnp.float32),
            pltpu.VMEM((NT, 1, vw), jnp.float32),
            pltpu.VMEM((vw, 1, NT), jnp.float32),
            pltpu.VMEM((N, 1, NT), jnp.float32),
            pltpu.VMEM((NT, 1, 128), jnp.float32),
            pltpu.VMEM((NT, 128), jnp.float32),
            pltpu.SemaphoreType.DMA((2,)),
        ],
    )
    return pl.pallas_call(
        functools.partial(_kernel, V),
        out_shape=jax.ShapeDtypeStruct((B, N, N), jnp.float32),
        grid_spec=grid_spec,
        compiler_params=pltpu.CompilerParams(
            dimension_semantics=("parallel", "arbitrary"),
            vmem_limit_bytes=56 * 1024 * 1024,
        ),
        name="gather_bilinear",
    )(words, W, wtail, wcol4, rootp)
